# baseline (speedup 1.0000x reference)
import jax
import jax.numpy as jnp
from jax.experimental import pallas as pl


def _rowdot_kernel(a_ref, b_ref, o_ref):
    o_ref[...] = jnp.sum(a_ref[...] * b_ref[...], axis=1)[None, None, :]


def kernel(inputs):
    gum = inputs[0]
    gim = inputs[1]
    n, k = gum.shape
    block = 4000
    out = pl.pallas_call(
        _rowdot_kernel,
        grid=(n // block,),
        in_specs=[
            pl.BlockSpec((block, k), lambda i: (i, 0)),
            pl.BlockSpec((block, k), lambda i: (i, 0)),
        ],
        out_specs=pl.BlockSpec((1, 1, block), lambda i: (i, 0, 0)),
        out_shape=jax.ShapeDtypeStruct((n // block, 1, block), gum.dtype),
    )(gum, gim)
    return out.reshape(n)


# speedup vs baseline: 1.7121x; 1.7121x over previous
import jax
import jax.numpy as jnp
from jax.experimental import pallas as pl


def _rowdot_kernel(in_ref, o_ref):
    a = in_ref[0]
    b = in_ref[1]
    o_ref[...] = jnp.sum(a * b, axis=1, keepdims=True)


def kernel(inputs):
    _, n, k = inputs.shape
    block = 4000
    out = pl.pallas_call(
        _rowdot_kernel,
        grid=(n // block,),
        in_specs=[pl.BlockSpec((2, block, k), lambda i: (0, i, 0))],
        out_specs=pl.BlockSpec((block, 1), lambda i: (i, 0)),
        out_shape=jax.ShapeDtypeStruct((n, 1), inputs.dtype),
    )(inputs)
    return out.reshape(n)


# speedup vs baseline: 2.9272x; 1.7097x over previous
import jax
import jax.numpy as jnp
from jax.experimental import pallas as pl


def _rowdot_kernel(in_ref, o_ref):
    a = in_ref[0]
    b = in_ref[1]
    c = (a * b).T
    o_ref[...] = jnp.sum(c, axis=0)[None, None, :]


def kernel(inputs):
    _, n, k = inputs.shape
    block = 4000
    out = pl.pallas_call(
        _rowdot_kernel,
        grid=(n // block,),
        in_specs=[pl.BlockSpec((2, block, k), lambda i: (0, i, 0))],
        out_specs=pl.BlockSpec((1, 1, block), lambda i: (i, 0, 0)),
        out_shape=jax.ShapeDtypeStruct((n // block, 1, block), inputs.dtype),
    )(inputs)
    return out.reshape(n)
